# baseline (speedup 1.0000x reference)
import functools

import jax
import jax.numpy as jnp
from jax import lax
from jax.experimental import pallas as pl
from jax.experimental.pallas import tpu as pltpu
from jax.experimental.pallas import tpu_sc as plsc

B = 8
NCH = 96
H = W = 384
P = H * W
NC = 2
NS = 16
NW = NC * NS
PW = NW // B
CHUNK = P // PW


def _body(in_hbm, lab_hbm, out_hbm, lab_v, buf):
    c = lax.axis_index("c")
    s = lax.axis_index("s")
    wid = s * NC + c
    b = wid // PW
    part = wid % PW

    pltpu.sync_copy(lab_hbm, lab_v)
    lv = lab_v[pl.ds(b, 16)]
    label_b = lv[0]

    row = (b * NCH + label_b) * PW + part
    pltpu.sync_copy(in_hbm.at[row], buf)
    pltpu.sync_copy(buf, out_hbm.at[wid])


def kernel(in_feat_map, labels):
    in2 = in_feat_map.reshape(B * NCH * PW, CHUNK)
    lab32 = jnp.zeros((32,), jnp.int32).at[:B].set(labels.astype(jnp.int32))

    mesh = plsc.VectorSubcoreMesh(core_axis_name="c", subcore_axis_name="s")
    run = functools.partial(
        pl.kernel,
        mesh=mesh,
        out_type=jax.ShapeDtypeStruct((NW, CHUNK), jnp.float32),
        scratch_types=[
            pltpu.VMEM((32,), jnp.int32),
            pltpu.VMEM((CHUNK,), jnp.float32),
        ],
    )(_body)
    out = run(in2, lab32)
    return out.reshape(B, H, W)


# speedup vs baseline: 19.2351x; 19.2351x over previous
import functools

import jax
import jax.numpy as jnp
from jax import lax
from jax.experimental import pallas as pl
from jax.experimental.pallas import tpu as pltpu
from jax.experimental.pallas import tpu_sc as plsc

B = 8
NCH = 96
H = W = 384
NC = 2
NS = 16
NW = NC * NS
PW = NW // B
ROWS = H // PW


def _body(in_hbm, lab_hbm, out_hbm, lab_v, buf):
    c = lax.axis_index("c")
    s = lax.axis_index("s")
    wid = s * NC + c
    b = wid // PW
    part = wid % PW

    pltpu.sync_copy(lab_hbm, lab_v)
    lv = lab_v[pl.ds(b, 16)]
    label_b = lv[0]

    r0 = part * ROWS
    pltpu.sync_copy(in_hbm.at[b, label_b, pl.ds(r0, ROWS)], buf)
    pltpu.sync_copy(buf, out_hbm.at[b, pl.ds(r0, ROWS)])


def kernel(in_feat_map, labels):
    lab32 = jnp.zeros((32,), jnp.int32).at[:B].set(labels.astype(jnp.int32))

    mesh = plsc.VectorSubcoreMesh(core_axis_name="c", subcore_axis_name="s")
    run = functools.partial(
        pl.kernel,
        mesh=mesh,
        out_type=jax.ShapeDtypeStruct((B, H, W), jnp.float32),
        scratch_types=[
            pltpu.VMEM((32,), jnp.int32),
            pltpu.VMEM((ROWS, W), jnp.float32),
        ],
    )(_body)
    return run(in_feat_map, lab32)


# speedup vs baseline: 20.1598x; 1.0481x over previous
import functools

import jax
import jax.numpy as jnp
from jax import lax
from jax.experimental import pallas as pl
from jax.experimental.pallas import tpu as pltpu
from jax.experimental.pallas import tpu_sc as plsc

B = 8
NCH = 96
H = W = 384
NC = 2
NS = 16
NW = NC * NS
PW = NW // B
ROWS = H // PW


def _body(in_hbm, lab_hbm, out_hbm, lab_v, buf):
    c = lax.axis_index("c")
    s = lax.axis_index("s")
    wid = s * NC + c
    b = wid // PW
    part = wid % PW

    pltpu.sync_copy(lab_hbm, lab_v.at[pl.ds(0, B)])
    lv = lab_v[pl.ds(b, 16)]
    label_b = lv[0]

    r0 = part * ROWS
    pltpu.sync_copy(in_hbm.at[b, label_b, pl.ds(r0, ROWS)], buf)
    pltpu.sync_copy(buf, out_hbm.at[b, pl.ds(r0, ROWS)])


def kernel(in_feat_map, labels):
    mesh = plsc.VectorSubcoreMesh(core_axis_name="c", subcore_axis_name="s")
    run = functools.partial(
        pl.kernel,
        mesh=mesh,
        out_type=jax.ShapeDtypeStruct((B, H, W), jnp.float32),
        scratch_types=[
            pltpu.VMEM((32,), jnp.int32),
            pltpu.VMEM((ROWS, W), jnp.float32),
        ],
    )(_body)
    return run(in_feat_map, labels.astype(jnp.int32))
